# baseline (speedup 1.0000x reference)
import jax
import jax.numpy as jnp
from jax import lax
from jax.experimental import pallas as pl


def _pos_embed_kernel(row_ref, col_ref, out_ref):
    col = col_ref[:32, :]
    row = row_ref[:32, :]
    k = lax.broadcasted_iota(jnp.int32, (32, 1024), 0)
    hw = lax.broadcasted_iota(jnp.int32, (32, 1024), 1)
    sel_w = (hw % 32 == k).astype(jnp.float32)
    sel_h = (hw // 32 == k).astype(jnp.float32)
    dn = (((0,), (0,)), ((), ()))
    x_part = lax.dot_general(col, sel_w, dn, preferred_element_type=jnp.float32)
    y_part = lax.dot_general(row, sel_h, dn, preferred_element_type=jnp.float32)
    out_ref[0, :256, :] = x_part
    out_ref[0, 256:, :] = y_part


def kernel(pixel_values, row_embeddings, column_embeddings):
    B = pixel_values.shape[0]
    H = pixel_values.shape[-2]
    W = pixel_values.shape[-1]
    D = row_embeddings.shape[-1]
    out = pl.pallas_call(
        _pos_embed_kernel,
        grid=(B,),
        in_specs=[
            pl.BlockSpec(row_embeddings.shape, lambda b: (0, 0)),
            pl.BlockSpec(column_embeddings.shape, lambda b: (0, 0)),
        ],
        out_specs=pl.BlockSpec((1, 2 * D, H * W), lambda b: (b, 0, 0)),
        out_shape=jax.ShapeDtypeStruct((B, 2 * D, H * W), jnp.float32),
    )(row_embeddings, column_embeddings)
    return out.reshape(B, 2 * D, H, W)


# speedup vs baseline: 1.0690x; 1.0690x over previous
import jax
import jax.numpy as jnp
from jax import lax
from jax.experimental import pallas as pl
from jax.experimental.pallas import tpu as pltpu

_B, _D, _H, _W = 8, 256, 32, 32


def _pos_embed_kernel(row_ref, col_ref, out_ref, plane_ref, sem):
    col = col_ref[:_W, :]
    row = row_ref[:_H, :]
    k = lax.broadcasted_iota(jnp.int32, (_W, _H * _W), 0)
    hw = lax.broadcasted_iota(jnp.int32, (_W, _H * _W), 1)
    sel_w = (hw % _W == k).astype(jnp.float32)
    sel_h = (hw // _W == k).astype(jnp.float32)
    dn = (((0,), (0,)), ((), ()))
    plane_ref[:_D, :] = lax.dot_general(
        col, sel_w, dn, preferred_element_type=jnp.float32)
    plane_ref[_D:, :] = lax.dot_general(
        row, sel_h, dn, preferred_element_type=jnp.float32)
    copies = [
        pltpu.make_async_copy(plane_ref, out_ref.at[b], sem) for b in range(_B)
    ]
    for c in copies:
        c.start()
    for c in copies:
        c.wait()


def kernel(pixel_values, row_embeddings, column_embeddings):
    B = pixel_values.shape[0]
    H = pixel_values.shape[-2]
    W = pixel_values.shape[-1]
    D = row_embeddings.shape[-1]
    out = pl.pallas_call(
        _pos_embed_kernel,
        in_specs=[
            pl.BlockSpec(memory_space=pltpu.VMEM),
            pl.BlockSpec(memory_space=pltpu.VMEM),
        ],
        out_specs=pl.BlockSpec(memory_space=pl.ANY),
        out_shape=jax.ShapeDtypeStruct((B, 2 * D, H * W), jnp.float32),
        scratch_shapes=[
            pltpu.VMEM((2 * D, H * W), jnp.float32),
            pltpu.SemaphoreType.DMA,
        ],
    )(row_embeddings, column_embeddings)
    return out.reshape(B, 2 * D, H, W)
